# baseline (speedup 1.0000x reference)
import jax
import jax.numpy as jnp
from jax.experimental import pallas as pl
from jax.experimental.pallas import tpu as pltpu

_BLOCK = 4096


def _vq_block_kernel(zea_ref, dmat_ref, cba_ref, out_ref):
    ze_aug = zea_ref[...]
    dmat = dmat_ref[...]
    cba = cba_ref[...]
    dist = jax.lax.dot_general(
        ze_aug, dmat, (((1,), (0,)), ((), ())), preferred_element_type=jnp.float32
    )
    min_d = jnp.min(dist, axis=1, keepdims=True)
    hot = jnp.where(dist == min_d, 1.0, 0.0)
    zq_aug = jax.lax.dot_general(
        hot, cba, (((1,), (0,)), ((), ())), preferred_element_type=jnp.float32
    )
    dim = zea_ref.shape[1] - 1
    out_ref[...] = zq_aug[:, :dim] / zq_aug[:, dim:]


@jax.jit
def kernel(x, code_book):
    b, h, w, c = x.shape
    n = b * h * w
    ze = x.reshape(n, c)
    ze_aug = jnp.pad(ze, ((0, 0), (0, 1)), constant_values=1.0)
    num_emb = code_book.shape[0]
    cb_norm = jnp.sum(code_book * code_book, axis=1)
    dmat = jnp.concatenate([-2.0 * code_book.T, cb_norm[None, :]], axis=0)
    cba = jnp.concatenate(
        [code_book, jnp.ones((num_emb, 1), code_book.dtype)], axis=1
    )
    zq = pl.pallas_call(
        _vq_block_kernel,
        grid=(n // _BLOCK,),
        in_specs=[
            pl.BlockSpec((_BLOCK, c + 1), lambda i: (i, 0)),
            pl.BlockSpec((c + 1, num_emb), lambda i: (0, 0)),
            pl.BlockSpec((num_emb, c + 1), lambda i: (0, 0)),
        ],
        out_specs=pl.BlockSpec((_BLOCK, c), lambda i: (i, 0)),
        out_shape=jax.ShapeDtypeStruct((n, c), x.dtype),
        compiler_params=pltpu.CompilerParams(
            dimension_semantics=("parallel",),
        ),
    )(ze_aug, dmat, cba)
    return zq.reshape(b, h, w, c)


# speedup vs baseline: 1.3893x; 1.3893x over previous
import jax
import jax.numpy as jnp
from jax.experimental import pallas as pl
from jax.experimental.pallas import tpu as pltpu

_BLOCK = 4096


def _vq_block_kernel(ze_ref, cbt_ref, cb_ref, out_ref):
    ze = ze_ref[...]
    cbt = cbt_ref[...]
    cb = cb_ref[...]
    cb_norm = jnp.sum(cbt * cbt, axis=0)[None, :]
    dist = cb_norm - 2.0 * jax.lax.dot_general(
        ze, cbt, (((1,), (0,)), ((), ())), preferred_element_type=jnp.float32
    )
    min_d = jnp.min(dist, axis=1, keepdims=True)
    hot = jnp.where(dist == min_d, 1.0, 0.0)
    count = jnp.sum(hot, axis=1, keepdims=True)
    zq = jax.lax.dot_general(
        hot, cb, (((1,), (0,)), ((), ())), preferred_element_type=jnp.float32
    )
    out_ref[...] = zq / count


@jax.jit
def kernel(x, code_book):
    b, h, w, c = x.shape
    n = b * h * w
    ze = x.reshape(n, c)
    num_emb = code_book.shape[0]
    zq = pl.pallas_call(
        _vq_block_kernel,
        grid=(n // _BLOCK,),
        in_specs=[
            pl.BlockSpec((_BLOCK, c), lambda i: (i, 0)),
            pl.BlockSpec((c, num_emb), lambda i: (0, 0)),
            pl.BlockSpec((num_emb, c), lambda i: (0, 0)),
        ],
        out_specs=pl.BlockSpec((_BLOCK, c), lambda i: (i, 0)),
        out_shape=jax.ShapeDtypeStruct((n, c), x.dtype),
        compiler_params=pltpu.CompilerParams(
            dimension_semantics=("parallel",),
        ),
    )(ze, code_book.T, code_book)
    return zq.reshape(b, h, w, c)
